# baseline (speedup 1.0000x reference)
import functools

import jax
import jax.numpy as jnp
import numpy as np
from jax import lax
from jax.experimental import pallas as pl
from jax.experimental.pallas import tpu as pltpu
from jax.experimental.pallas import tpu_sc as plsc

N = 10000
E = 320000
D_IN = 128
D1 = 16
D2 = 16
NC = 2
NS = 16
NW = NC * NS
B = 125
K = 80
NPAD = 10112
RPT = NPAD // NS
NP128 = NPAD * D1 // 128
N128 = N * D1 // 128

f32 = jnp.float32
i32 = jnp.int32

_mesh = plsc.VectorSubcoreMesh(core_axis_name="c", subcore_axis_name="s")


_LAG = 8

@functools.partial(
    pl.kernel,
    out_type=jax.ShapeDtypeStruct((NC, NPAD, D1), f32),
    mesh=_mesh,
    compiler_params=pltpu.CompilerParams(use_tc_tiling_on_sc=False),
    scratch_types=[
        pltpu.VMEM((K, B), i32),
        pltpu.VMEM((B, D1), f32),
        pltpu.VMEM_SHARED((NPAD, D1), f32),
        pltpu.SemaphoreType.DMA,
    ],
)
def _sc_degree(ei_hbm, ones_hbm, zeros_hbm, out_hbm, cols_v, ones_v, acc_sh, sem):
    c = lax.axis_index("c")
    s = lax.axis_index("s")
    wid = s * NC + c
    pltpu.sync_copy(ei_hbm.at[1, wid], cols_v)
    pltpu.sync_copy(ones_hbm, ones_v)
    pltpu.sync_copy(zeros_hbm.at[pl.ds(s * RPT, RPT)], acc_sh.at[pl.ds(s * RPT, RPT)])
    plsc.subcore_barrier()

    pend = []
    for j in range(K):
        pend.append(pltpu.async_copy(ones_v, acc_sh.at[cols_v.at[j]], sem, add=True))
        if len(pend) > _LAG:
            pend.pop(0).wait()
    for d in pend:
        d.wait()
    plsc.subcore_barrier()
    pltpu.sync_copy(acc_sh.at[pl.ds(s * RPT, RPT)], out_hbm.at[c, pl.ds(s * RPT, RPT)])


_G = 6
_LEAD = 3


def _make_sc_edge_pass(d):
    @functools.partial(
        pl.kernel,
        out_type=jax.ShapeDtypeStruct((NC, NPAD, d), f32),
        mesh=_mesh,
        compiler_params=pltpu.CompilerParams(use_tc_tiling_on_sc=False),
        scratch_types=[
            pltpu.VMEM((K, B), i32),
            pltpu.VMEM((K, B), i32),
            pltpu.VMEM_SHARED((NPAD, d), f32),
        ]
        + [pltpu.VMEM((B, d), f32)] * _G
        + [pltpu.SemaphoreType.DMA] * (2 * _G),
    )
    def edge_pass(p_hbm, ei_hbm, zeros_hbm, out_hbm,
                  rows_v, cols_v, acc_sh, *rest):
        bufs = rest[:_G]
        gsem = rest[_G:2 * _G]
        ssem = rest[2 * _G:]
        c = lax.axis_index("c")
        s = lax.axis_index("s")
        wid = s * NC + c
        pltpu.sync_copy(ei_hbm.at[0, wid], rows_v)
        pltpu.sync_copy(ei_hbm.at[1, wid], cols_v)
        pltpu.sync_copy(zeros_hbm.at[pl.ds(s * RPT, RPT)], acc_sh.at[pl.ds(s * RPT, RPT)])
        plsc.subcore_barrier()

        gat = {}
        sca = {}

        def scatter(t):
            ts = t % _G
            gat.pop(ts).wait()
            sca[ts] = pltpu.async_copy(bufs[ts], acc_sh.at[cols_v.at[t]],
                                       ssem[ts], add=True)

        for j in range(K):
            slot = j % _G
            if slot in sca:
                sca.pop(slot).wait()
            gat[slot] = pltpu.async_copy(p_hbm.at[rows_v.at[j]], bufs[slot],
                                         gsem[slot])
            if j >= _LEAD:
                scatter(j - _LEAD)
        for t in range(K - _LEAD, K):
            scatter(t)
        for desc in sca.values():
            desc.wait()
        plsc.subcore_barrier()
        pltpu.sync_copy(acc_sh.at[pl.ds(s * RPT, RPT)], out_hbm.at[c, pl.ds(s * RPT, RPT)])

    return edge_pass


_sc_edge_16 = _make_sc_edge_pass(D1)


def _lin1_body(x8_ref, w_ref, b_ref, dg_ref, dis_ref, p_ref):
    deg = dg_ref[0] + dg_ref[1] + 1.0
    dis = lax.rsqrt(deg)
    dis_ref[...] = dis
    hp = jnp.dot(x8_ref[...], w_ref[...], preferred_element_type=f32) + b_ref[...]
    hp = jnp.concatenate([hp, jnp.zeros((NP128 - N128, 128), f32)])
    p_ref[...] = dis * hp


def _tc_lin1(x8, W1blk, b1blk, deg2p):
    return pl.pallas_call(
        _lin1_body,
        in_specs=[
            pl.BlockSpec((N128, 8 * D_IN), lambda: (0, 0)),
            pl.BlockSpec((8 * D_IN, 128), lambda: (0, 0)),
            pl.BlockSpec((1, 128), lambda: (0, 0)),
            pl.BlockSpec((2, NP128, 128), lambda: (0, 0, 0)),
        ],
        out_specs=[pl.BlockSpec((NP128, 128), lambda: (0, 0))] * 2,
        out_shape=[jax.ShapeDtypeStruct((NP128, 128), f32)] * 2,
    )(x8, W1blk, b1blk, deg2p)


def _mid_body(a_ref, p1_ref, dis_ref, w_ref, b_ref, p2_ref):
    dis = dis_ref[...]
    t = jnp.maximum(dis * (a_ref[0] + a_ref[1] + p1_ref[...]), 0.0)
    h2 = jnp.dot(t, w_ref[...], preferred_element_type=f32) + b_ref[...]
    p2_ref[...] = dis * h2


def _tc_mid(acc1p, p1p, disp, W2blk, b2blk):
    return pl.pallas_call(
        _mid_body,
        in_specs=[
            pl.BlockSpec((2, NP128, 128), lambda: (0, 0, 0)),
            pl.BlockSpec((NP128, 128), lambda: (0, 0)),
            pl.BlockSpec((NP128, 128), lambda: (0, 0)),
            pl.BlockSpec((128, 128), lambda: (0, 0)),
            pl.BlockSpec((1, 128), lambda: (0, 0)),
        ],
        out_specs=pl.BlockSpec((NP128, 128), lambda: (0, 0)),
        out_shape=jax.ShapeDtypeStruct((NP128, 128), f32),
    )(acc1p, p1p, disp, W2blk, b2blk)


def _out_body(a_ref, p2_ref, dis_ref, sum_ref, o_ref):
    oP = jnp.maximum(dis_ref[...] * (a_ref[0] + a_ref[1] + p2_ref[...]), 0.0)
    lane = lax.broadcasted_iota(i32, oP.shape, 1)
    oP = jnp.where(lane % D2 < 7, oP, -1e30)
    m = jnp.max(oP, axis=1, keepdims=True)
    e = jnp.exp(oP - m)
    s = jnp.dot(e, sum_ref[...], preferred_element_type=f32)
    o_ref[...] = (oP - m) - jnp.log(s)


def _tc_out(acc2p, p2p, disp, sumblk):
    return pl.pallas_call(
        _out_body,
        in_specs=[
            pl.BlockSpec((2, NP128, 128), lambda: (0, 0, 0)),
            pl.BlockSpec((NP128, 128), lambda: (0, 0)),
            pl.BlockSpec((NP128, 128), lambda: (0, 0)),
            pl.BlockSpec((128, 128), lambda: (0, 0)),
        ],
        out_specs=pl.BlockSpec((NP128, 128), lambda: (0, 0)),
        out_shape=jax.ShapeDtypeStruct((NP128, 128), f32),
    )(acc2p, p2p, disp, sumblk)


def kernel(x, edge_index, W1, b1, W2, b2):
    ei4 = edge_index.astype(i32).reshape(2, NW, K, B)
    ones_b = np.ones((B, D1), np.float32)
    zeros_acc = np.zeros((NPAD, D1), np.float32)
    eye8 = jnp.eye(8, dtype=f32)
    W1blk = jnp.kron(eye8, W1)
    b1blk = jnp.tile(b1, 8).reshape(1, 128)
    W2p = jnp.pad(W2, ((0, 0), (0, D2 - 7)))
    W2blk = jnp.kron(eye8, W2p)
    b2blk = jnp.tile(jnp.pad(b2, (0, D2 - 7)), 8).reshape(1, 128)
    sumblk = jnp.kron(eye8, jnp.ones((D2, D2), f32))
    x8 = x.reshape(N128, 8 * D_IN)

    deg2 = _sc_degree(ei4, ones_b, zeros_acc)
    x8, deg2 = lax.optimization_barrier((x8, deg2))
    disp, p1p = _tc_lin1(x8, W1blk, b1blk, deg2.reshape(NC, NP128, 128))
    acc1 = _sc_edge_16(p1p.reshape(NPAD, D1), ei4, zeros_acc)
    p2p = _tc_mid(acc1.reshape(NC, NP128, 128), p1p, disp, W2blk, b2blk)
    acc2 = _sc_edge_16(p2p.reshape(NPAD, D1), ei4, zeros_acc)
    o = _tc_out(acc2.reshape(NC, NP128, 128), p2p, disp, sumblk)
    return o.reshape(NPAD, D1)[:N, :7]


# speedup vs baseline: 1.1262x; 1.1262x over previous
import functools

import jax
import jax.numpy as jnp
import numpy as np
from jax import lax
from jax.experimental import pallas as pl
from jax.experimental.pallas import tpu as pltpu
from jax.experimental.pallas import tpu_sc as plsc

N = 10000
E = 320000
D_IN = 128
D1 = 16
D2 = 16
NC = 2
NS = 16
NW = NC * NS
B = 125
K = 80
NPAD = 10112
RPT = NPAD // NS
NP128 = NPAD * D1 // 128
N128 = N * D1 // 128

f32 = jnp.float32
i32 = jnp.int32

_mesh = plsc.VectorSubcoreMesh(core_axis_name="c", subcore_axis_name="s")


_LAG = 8

@functools.partial(
    pl.kernel,
    out_type=jax.ShapeDtypeStruct((NC, NPAD, D1), f32),
    mesh=_mesh,
    compiler_params=pltpu.CompilerParams(use_tc_tiling_on_sc=False),
    scratch_types=[
        pltpu.VMEM((K, B), i32),
        pltpu.VMEM((B, D1), f32),
        pltpu.VMEM_SHARED((NPAD, D1), f32),
        pltpu.SemaphoreType.DMA,
    ],
)
def _sc_degree(ei_hbm, ones_hbm, zeros_hbm, out_hbm, cols_v, ones_v, acc_sh, sem):
    c = lax.axis_index("c")
    s = lax.axis_index("s")
    wid = s * NC + c
    pltpu.sync_copy(ei_hbm.at[1, wid], cols_v)
    pltpu.sync_copy(ones_hbm, ones_v)
    pltpu.sync_copy(zeros_hbm.at[pl.ds(s * RPT, RPT)], acc_sh.at[pl.ds(s * RPT, RPT)])
    plsc.subcore_barrier()

    pend = []
    for j in range(K):
        pend.append(pltpu.async_copy(ones_v, acc_sh.at[cols_v.at[j]], sem, add=True))
        if len(pend) > _LAG:
            pend.pop(0).wait()
    for d in pend:
        d.wait()
    plsc.subcore_barrier()
    pltpu.sync_copy(acc_sh.at[pl.ds(s * RPT, RPT)], out_hbm.at[c, pl.ds(s * RPT, RPT)])


_G = 6
_LEAD = 3


def _make_sc_edge_pass(d):
    @functools.partial(
        pl.kernel,
        out_type=jax.ShapeDtypeStruct((NC, NPAD, d), f32),
        mesh=_mesh,
        compiler_params=pltpu.CompilerParams(use_tc_tiling_on_sc=False),
        scratch_types=[
            pltpu.VMEM((K, B), i32),
            pltpu.VMEM((K, B), i32),
            pltpu.VMEM_SHARED((NPAD, d), f32),
        ]
        + [pltpu.VMEM((B, d), f32)] * _G
        + [pltpu.SemaphoreType.DMA] * (2 * _G),
    )
    def edge_pass(p_hbm, ei_hbm, zeros_hbm, out_hbm,
                  rows_v, cols_v, acc_sh, *rest):
        bufs = rest[:_G]
        gsem = rest[_G:2 * _G]
        ssem = rest[2 * _G:]
        c = lax.axis_index("c")
        s = lax.axis_index("s")
        wid = s * NC + c
        pltpu.sync_copy(ei_hbm.at[0, wid], rows_v)
        pltpu.sync_copy(ei_hbm.at[1, wid], cols_v)
        pltpu.sync_copy(zeros_hbm.at[pl.ds(s * RPT, RPT)], acc_sh.at[pl.ds(s * RPT, RPT)])
        plsc.subcore_barrier()

        gat = {}
        sca = {}

        def scatter(t):
            ts = t % _G
            gat.pop(ts).wait()
            sca[ts] = pltpu.async_copy(bufs[ts], acc_sh.at[cols_v.at[t]],
                                       ssem[ts], add=True)

        for j in range(K):
            slot = j % _G
            if slot in sca:
                sca.pop(slot).wait()
            gat[slot] = pltpu.async_copy(p_hbm.at[rows_v.at[j]], bufs[slot],
                                         gsem[slot])
            if j >= _LEAD:
                scatter(j - _LEAD)
        for t in range(K - _LEAD, K):
            scatter(t)
        for desc in sca.values():
            desc.wait()
        plsc.subcore_barrier()
        pltpu.sync_copy(acc_sh.at[pl.ds(s * RPT, RPT)], out_hbm.at[c, pl.ds(s * RPT, RPT)])

    return edge_pass


_sc_edge_16 = _make_sc_edge_pass(D1)


def _lin1_body(x8_ref, w_ref, b_ref, dg_ref, dis_ref, p_ref):
    deg = dg_ref[0] + dg_ref[1] + 1.0
    dis = lax.rsqrt(deg)
    dis_ref[...] = dis
    hp = jnp.dot(x8_ref[...], w_ref[...], preferred_element_type=f32) + b_ref[...]
    hp = jnp.concatenate([hp, jnp.zeros((NP128 - N128, 128), f32)])
    p_ref[...] = dis * hp


def _tc_lin1(x8, W1blk, b1blk, deg2p):
    return pl.pallas_call(
        _lin1_body,
        in_specs=[
            pl.BlockSpec((N128, 8 * D_IN), lambda: (0, 0)),
            pl.BlockSpec((8 * D_IN, 128), lambda: (0, 0)),
            pl.BlockSpec((1, 128), lambda: (0, 0)),
            pl.BlockSpec((2, NP128, 128), lambda: (0, 0, 0)),
        ],
        out_specs=[pl.BlockSpec((NP128, 128), lambda: (0, 0))] * 2,
        out_shape=[jax.ShapeDtypeStruct((NP128, 128), f32)] * 2,
    )(x8, W1blk, b1blk, deg2p)


def _mid_body(a_ref, p1_ref, dis_ref, w_ref, b_ref, p2_ref):
    dis = dis_ref[...]
    t = jnp.maximum(dis * (a_ref[0] + a_ref[1] + p1_ref[...]), 0.0)
    h2 = jnp.dot(t, w_ref[...], preferred_element_type=f32) + b_ref[...]
    p2_ref[...] = dis * h2


def _tc_mid(acc1p, p1p, disp, W2blk, b2blk):
    return pl.pallas_call(
        _mid_body,
        in_specs=[
            pl.BlockSpec((2, NP128, 128), lambda: (0, 0, 0)),
            pl.BlockSpec((NP128, 128), lambda: (0, 0)),
            pl.BlockSpec((NP128, 128), lambda: (0, 0)),
            pl.BlockSpec((128, 128), lambda: (0, 0)),
            pl.BlockSpec((1, 128), lambda: (0, 0)),
        ],
        out_specs=pl.BlockSpec((NP128, 128), lambda: (0, 0)),
        out_shape=jax.ShapeDtypeStruct((NP128, 128), f32),
    )(acc1p, p1p, disp, W2blk, b2blk)


def _out_body(a_ref, p2_ref, dis_ref, sum_ref, o_ref):
    oP = jnp.maximum(dis_ref[...] * (a_ref[0] + a_ref[1] + p2_ref[...]), 0.0)
    lane = lax.broadcasted_iota(i32, oP.shape, 1)
    oP = jnp.where(lane % D2 < 7, oP, -1e30)
    m = jnp.max(oP, axis=1, keepdims=True)
    e = jnp.exp(oP - m)
    s = jnp.dot(e, sum_ref[...], preferred_element_type=f32)
    o_ref[...] = (oP - m) - jnp.log(s)


def _tc_out(acc2p, p2p, disp, sumblk):
    return pl.pallas_call(
        _out_body,
        in_specs=[
            pl.BlockSpec((2, NP128, 128), lambda: (0, 0, 0)),
            pl.BlockSpec((NP128, 128), lambda: (0, 0)),
            pl.BlockSpec((NP128, 128), lambda: (0, 0)),
            pl.BlockSpec((128, 128), lambda: (0, 0)),
        ],
        out_specs=pl.BlockSpec((NP128, 128), lambda: (0, 0)),
        out_shape=jax.ShapeDtypeStruct((NP128, 128), f32),
    )(acc2p, p2p, disp, sumblk)


def kernel(x, edge_index, W1, b1, W2, b2):
    ei4 = edge_index.astype(i32).reshape(2, NW, K, B)
    ones_b = np.ones((B, D1), np.float32)
    zeros_acc = np.zeros((NPAD, D1), np.float32)
    eye8 = jnp.eye(8, dtype=f32)
    W1blk = jnp.kron(eye8, W1)
    b1blk = jnp.tile(b1, 8).reshape(1, 128)
    W2p = jnp.pad(W2, ((0, 0), (0, D2 - 7)))
    W2blk = jnp.kron(eye8, W2p)
    b2blk = jnp.tile(jnp.pad(b2, (0, D2 - 7)), 8).reshape(1, 128)
    sumblk = jnp.kron(eye8, jnp.ones((D2, D2), f32))
    x8 = x.reshape(N128, 8 * D_IN)

    deg2 = _sc_degree(ei4, ones_b, zeros_acc)
    disp, p1p = _tc_lin1(x8, W1blk, b1blk, deg2.reshape(NC, NP128, 128))
    acc1 = _sc_edge_16(p1p.reshape(NPAD, D1), ei4, zeros_acc)
    p2p = _tc_mid(acc1.reshape(NC, NP128, 128), p1p, disp, W2blk, b2blk)
    acc2 = _sc_edge_16(p2p.reshape(NPAD, D1), ei4, zeros_acc)
    o = _tc_out(acc2.reshape(NC, NP128, 128), p2p, disp, sumblk)
    return o[:N128].reshape(N, D1)[:, :7]
